# baseline (speedup 1.0000x reference)
import functools
import math
import jax
import jax.numpy as jnp
from jax import lax
from jax.experimental import pallas as pl
from jax.experimental.pallas import tpu as pltpu
from jax.experimental.pallas import tpu_sc as plsc

_B, _A, _N, _H, _K = 16, 4, 8192, 64, 32
_R = _B * _A
_NORM = 1.0 / math.sqrt(_H)
_HP = jax.lax.Precision.HIGHEST


def _pass1_body(vs_ref, ve_ref, dead_ref, wq_ref, wk_ref, wv_ref, motw_ref,
                motb_ref, compat_ref, vm_ref, wide_ref):
    vsr = vs_ref[0]
    q = jax.lax.dot(vsr, wq_ref[...])
    qb = q.astype(jnp.bfloat16)
    vemat = ve_ref[0, 0]
    wide_ref[0] = jnp.concatenate([vemat[:_N // 2], vemat[_N // 2:]], axis=1)
    veb = vemat.astype(jnp.bfloat16)
    kb = jax.lax.dot(veb, wk_ref[...],
                     preferred_element_type=jnp.float32
                     ).astype(jnp.bfloat16)
    c = _NORM * jax.lax.dot_general(qb, kb, (((1,), (1,)), ((), ())),
                                    preferred_element_type=jnp.float32)
    c = jnp.where(dead_ref[0] != 0, -jnp.inf, c)
    compat_ref[0] = c
    m = jnp.max(c)
    e = jnp.exp(c - jnp.maximum(m, -1e30))
    s = jnp.sum(e)
    w = jax.lax.dot(e.astype(jnp.bfloat16), veb,
                    preferred_element_type=jnp.float32)
    inv = jnp.where(s > 0, 1.0 / s, 0.0)
    va = jax.lax.dot(w * inv, wv_ref[...], precision=_HP)
    vm_in = jnp.concatenate([vsr, va], axis=1)
    vm = jax.lax.dot_general(vm_in, motw_ref[...], (((1,), (1,)), ((), ())),
                             precision=_HP) + motb_ref[...]
    vm_ref[0] = jnp.maximum(vm, 0.0)


def _pass1(vs3, ve3, dead3, Wq, Wk, Wv, mot_w, mot_b2):
    return pl.pallas_call(
        _pass1_body,
        grid=(_R,),
        in_specs=[
            pl.BlockSpec((1, 1, _H), lambda r: (r, 0, 0)),
            pl.BlockSpec((1, 1, _N, _H), lambda r: (r // _A, r % _A, 0, 0)),
            pl.BlockSpec((1, 1, _N), lambda r: (r, 0, 0)),
            pl.BlockSpec((_H, _H), lambda r: (0, 0)),
            pl.BlockSpec((_H, _H), lambda r: (0, 0)),
            pl.BlockSpec((_H, _H), lambda r: (0, 0)),
            pl.BlockSpec((_H, 2 * _H), lambda r: (0, 0)),
            pl.BlockSpec((1, _H), lambda r: (0, 0)),
        ],
        out_specs=[
            pl.BlockSpec((1, 1, _N), lambda r: (r, 0, 0)),
            pl.BlockSpec((1, 1, _H), lambda r: (r, 0, 0)),
            pl.BlockSpec((1, _N // 2, 2 * _H), lambda r: (r, 0, 0)),
        ],
        out_shape=[
            jax.ShapeDtypeStruct((_R, 1, _N), jnp.float32),
            jax.ShapeDtypeStruct((_R, 1, _H), jnp.float32),
            jax.ShapeDtypeStruct((_R, _N // 2, 2 * _H), jnp.float32),
        ],
    )(vs3, ve3, dead3, Wq, Wk, Wv, mot_w, mot_b2)


_NV = _N // 16
_MAXI = 2**31 - 1
_NEG = float("-inf")


_GD = lax.GatherDimensionNumbers(offset_dims=(), collapsed_slice_dims=(0,),
                                 start_index_map=(0,))


def _lane(x, j):
    return lax.gather(x, jnp.full((16, 1), j, jnp.int32), _GD,
                      slice_sizes=(1,),
                      mode=lax.GatherScatterMode.PROMISE_IN_BOUNDS)


def _topk_row(compat_hbm, ve_hbm, out_hbm, crow, cvals, cidx, tkg, thalf,
              wide, rows, sem, row):
    pltpu.sync_copy(compat_hbm.at[pl.ds(row * _N, _N)], crow)
    lanes = lax.iota(jnp.int32, 16)
    ninf = jnp.full((16,), _NEG, jnp.float32)
    maxi = jnp.full((16,), _MAXI, jnp.int32)

    def t2_body(i, carry):
        t1, t2 = carry
        v = crow[pl.ds(i * 16, 16)]
        m = v > t1
        hi = jnp.where(m, v, t1)
        lo = jnp.where(m, t1, v)
        return hi, jnp.maximum(t2, lo)

    _, t2 = lax.fori_loop(0, _NV, t2_body, (ninf, ninf))
    srt2, _ = plsc.sort_key_val(t2, t2)
    t = _lane(srt2, 0)

    def comp_body(i, off):
        v = crow[pl.ds(i * 16, 16)]
        mask = v >= t
        pc = plsc.cumsum(mask.astype(jnp.int32))
        pos = jnp.where(mask, off + pc - 1, 0)
        plsc.store_scatter(cvals, [pos], v, mask=mask)
        plsc.store_scatter(cidx, [pos], lanes + i * 16, mask=mask)
        return off + _lane(pc, 15)

    off = lax.fori_loop(0, _NV, comp_body, jnp.zeros((16,), jnp.int32))
    plsc.store_scatter(cvals, [off + lanes], ninf)

    def ext_body(k, _):
        def scan_cond(c):
            return jnp.any(c[0] * 16 < off)

        def scan_step(c):
            j, bv, bi, bp = c
            v = cvals[pl.ds(j * 16, 16)]
            iv = cidx[pl.ds(j * 16, 16)]
            pos = lanes + j * 16
            m = (v > bv) | ((v == bv) & (iv < bi))
            return (j + 1, jnp.where(m, v, bv), jnp.where(m, iv, bi),
                    jnp.where(m, pos, bp))

        _, bv, bi, bp = lax.while_loop(
            scan_cond, scan_step,
            (0, ninf, maxi, jnp.zeros((16,), jnp.int32)))
        srtv, _ = plsc.sort_key_val(bv, bv)
        mval = _lane(srtv, 15)
        ci = jnp.where(bv == mval, bi, maxi)
        srti, _ = plsc.sort_key_val(ci, ci)
        mi = _lane(srti, 0)
        winner = ci == mi
        plsc.store_scatter(cvals, [bp], ninf, mask=winner)
        kvec = jnp.full((16,), 1, jnp.int32) * k
        lane0 = lanes == 0
        plsc.store_scatter(tkg, [kvec],
                           (mi & (_N // 2 - 1)) + row * (_N // 2), mask=lane0)
        plsc.store_scatter(thalf, [kvec], lax.shift_right_logical(mi, 12),
                           mask=lane0)
        return 0

    lax.fori_loop(0, _K, ext_body, 0)
    pltpu.async_copy(ve_hbm.at[tkg], wide, sem).wait()
    h0 = thalf[pl.ds(0, 16)]
    h1 = thalf[pl.ds(16, 16)]
    for j in range(_K * _H // 16):
        kj = j // (_H // 16)
        half = _lane(h0 if kj < 16 else h1, kj % 16)
        rowv = jnp.full((16,), kj, jnp.int32)
        colv = half * _H + (lanes + (j % (_H // 16)) * 16)
        rows[pl.ds(j * 16, 16)] = plsc.load_gather(wide, [rowv, colv])
    pltpu.sync_copy(rows, out_hbm.at[pl.ds(row * _K * _H, _K * _H)])


def _sc_topk_gather(compat, ve2):
    mesh = plsc.VectorSubcoreMesh(core_axis_name="c", subcore_axis_name="s")

    @functools.partial(
        pl.kernel, mesh=mesh,
        compiler_params=pltpu.CompilerParams(needs_layout_passes=False),
        out_type=jax.ShapeDtypeStruct((_R * _K * _H,), jnp.float32),
        scratch_types=[
            pltpu.VMEM((_N,), jnp.float32),
            pltpu.VMEM((_N + 16,), jnp.float32),
            pltpu.VMEM((_N + 16,), jnp.int32),
            pltpu.VMEM((_K,), jnp.int32),
            pltpu.VMEM((_K,), jnp.int32),
            pltpu.VMEM((_K, 2 * _H), jnp.float32),
            pltpu.VMEM((_K * _H,), jnp.float32),
            pltpu.SemaphoreType.DMA,
        ],
    )
    def k(compat_hbm, ve_hbm, out_hbm, crow, cvals, cidx, tkg, thalf, wide,
          rows, sem):
        wid = lax.axis_index("s") * 2 + lax.axis_index("c")
        for rr in range(2):
            _topk_row(compat_hbm, ve_hbm, out_hbm, crow, cvals, cidx, tkg,
                      thalf, wide, rows, sem, wid * 2 + rr)

    return k(compat, ve2)


def _final_body(vs_ref, g_ref, fvs_ref, fve_ref, fb_ref, out_ref):
    acc = jax.lax.dot_general(vs_ref[...], fvs_ref[...],
                              (((1,), (1,)), ((), ())), precision=_HP)
    acc = acc + jax.lax.dot_general(g_ref[...], fve_ref[...],
                                    (((1,), (1,)), ((), ())), precision=_HP)
    out_ref[...] = jnp.maximum(acc + fb_ref[...], 0.0)


def _final(vs2, g2, fwd_vs, fwd_ve, fwd_b2):
    return pl.pallas_call(
        _final_body,
        out_shape=jax.ShapeDtypeStruct((_R, _H), jnp.float32),
    )(vs2, g2, fwd_vs, fwd_ve, fwd_b2)


def kernel(vs, ve, ve_dead, Wq, Wk, Wv, mot_w, mot_b, fwd_w, fwd_b):
    vs3 = vs.reshape(_R, 1, _H)
    dead3 = ve_dead.reshape(_R, 1, _N).astype(jnp.float32)
    compat3, vm3, wide3 = _pass1(vs3, ve, dead3, Wq, Wk.astype(jnp.bfloat16),
                                 Wv, mot_w, mot_b.reshape(1, _H))
    gathered = _sc_topk_gather(compat3.reshape(_R * _N),
                               wide3.reshape(_R * _N // 2, 2 * _H))

    g2 = gathered.reshape(_R, _K * _H)
    v_C = _final(vs3.reshape(_R, _H), g2, fwd_w[:, :_H], fwd_w[:, _H:],
                 fwd_b.reshape(1, _H))
    return (v_C.reshape(_B, _A, _H), vm3.reshape(_B, _A, _H))


# speedup vs baseline: 1.2584x; 1.2584x over previous
import functools
import math
import jax
import jax.numpy as jnp
from jax import lax
from jax.experimental import pallas as pl
from jax.experimental.pallas import tpu as pltpu
from jax.experimental.pallas import tpu_sc as plsc

_B, _A, _N, _H, _K = 16, 4, 8192, 64, 32
_R = _B * _A
_NORM = 1.0 / math.sqrt(_H)
_HP = jax.lax.Precision.HIGHEST


def _pass1_body(vs_ref, ve_ref, dead_ref, wq_ref, wk_ref, wv_ref, motw_ref,
                motb_ref, compat_ref, vm_ref, wide_ref):
    vsr = vs_ref[0]
    q = jax.lax.dot(vsr, wq_ref[...])
    qb = q.astype(jnp.bfloat16)
    vemat = ve_ref[0]
    wide_ref[...] = jnp.concatenate([vemat[:_N // 2], vemat[_N // 2:]],
                                    axis=1)
    veb = vemat.astype(jnp.bfloat16)
    kb = jax.lax.dot(veb, wk_ref[...],
                     preferred_element_type=jnp.float32
                     ).astype(jnp.bfloat16)
    c = _NORM * jax.lax.dot_general(qb, kb, (((1,), (1,)), ((), ())),
                                    preferred_element_type=jnp.float32)
    c = jnp.where(dead_ref[0] != 0, -jnp.inf, c)
    compat_ref[0] = c
    m = jnp.max(c)
    e = jnp.exp(c - jnp.maximum(m, -1e30))
    s = jnp.sum(e)
    w = jax.lax.dot(e.astype(jnp.bfloat16), veb,
                    preferred_element_type=jnp.float32)
    inv = jnp.where(s > 0, 1.0 / s, 0.0)
    va = jax.lax.dot(w * inv, wv_ref[...], precision=_HP)
    vm_in = jnp.concatenate([vsr, va], axis=1)
    vm = jax.lax.dot_general(vm_in, motw_ref[...], (((1,), (1,)), ((), ())),
                             precision=_HP) + motb_ref[...]
    vm_ref[0] = jnp.maximum(vm, 0.0)


def _pass1(vs3, ve3, dead3, Wq, Wk, Wv, mot_w, mot_b2):
    return pl.pallas_call(
        _pass1_body,
        grid=(_R,),
        in_specs=[
            pl.BlockSpec((1, 1, _H), lambda r: (r, 0, 0)),
            pl.BlockSpec((1, _N, _H), lambda r: (r, 0, 0)),
            pl.BlockSpec((1, 1, _N), lambda r: (r, 0, 0)),
            pl.BlockSpec((_H, _H), lambda r: (0, 0)),
            pl.BlockSpec((_H, _H), lambda r: (0, 0)),
            pl.BlockSpec((_H, _H), lambda r: (0, 0)),
            pl.BlockSpec((_H, 2 * _H), lambda r: (0, 0)),
            pl.BlockSpec((1, _H), lambda r: (0, 0)),
        ],
        out_specs=[
            pl.BlockSpec((1, 1, _N), lambda r: (r, 0, 0)),
            pl.BlockSpec((1, 1, _H), lambda r: (r, 0, 0)),
            pl.BlockSpec((_N // 2, 2 * _H), lambda r: (r, 0)),
        ],
        out_shape=[
            jax.ShapeDtypeStruct((_R, 1, _N), jnp.float32),
            jax.ShapeDtypeStruct((_R, 1, _H), jnp.float32),
            jax.ShapeDtypeStruct((_R * _N // 2, 2 * _H), jnp.float32),
        ],
    )(vs3, ve3, dead3, Wq, Wk, Wv, mot_w, mot_b2)


_NV = _N // 16
_MAXI = 2**31 - 1
_NEG = float("-inf")


_GD = lax.GatherDimensionNumbers(offset_dims=(), collapsed_slice_dims=(0,),
                                 start_index_map=(0,))


def _lane(x, j):
    return lax.gather(x, jnp.full((16, 1), j, jnp.int32), _GD,
                      slice_sizes=(1,),
                      mode=lax.GatherScatterMode.PROMISE_IN_BOUNDS)


def _topk_row(compat_hbm, ve_hbm, out_hbm, crow, cvals, cidx, tkg, thalf,
              wide, rows, sem, row):
    pltpu.sync_copy(compat_hbm.at[pl.ds(row * _N, _N)], crow)
    lanes = lax.iota(jnp.int32, 16)
    ninf = jnp.full((16,), _NEG, jnp.float32)
    maxi = jnp.full((16,), _MAXI, jnp.int32)

    def t2_body(i, carry):
        t1, t2 = carry
        v = crow[pl.ds(i * 16, 16)]
        m = v > t1
        hi = jnp.where(m, v, t1)
        lo = jnp.where(m, t1, v)
        return hi, jnp.maximum(t2, lo)

    _, t2 = lax.fori_loop(0, _NV, t2_body, (ninf, ninf))
    srt2, _ = plsc.sort_key_val(t2, t2)
    t = _lane(srt2, 0)

    def comp_body(i, off):
        v = crow[pl.ds(i * 16, 16)]
        mask = v >= t
        pc = plsc.cumsum(mask.astype(jnp.int32))
        pos = jnp.where(mask, off + pc - 1, 0)
        plsc.store_scatter(cvals, [pos], v, mask=mask)
        plsc.store_scatter(cidx, [pos], lanes + i * 16, mask=mask)
        return off + _lane(pc, 15)

    off = lax.fori_loop(0, _NV, comp_body, jnp.zeros((16,), jnp.int32))
    plsc.store_scatter(cvals, [off + lanes], ninf)

    def ext_body(k, _):
        def scan_cond(c):
            return jnp.any(c[0] * 16 < off)

        def scan_step(c):
            j, bv, bi, bp = c
            v = cvals[pl.ds(j * 16, 16)]
            iv = cidx[pl.ds(j * 16, 16)]
            pos = lanes + j * 16
            m = (v > bv) | ((v == bv) & (iv < bi))
            return (j + 1, jnp.where(m, v, bv), jnp.where(m, iv, bi),
                    jnp.where(m, pos, bp))

        _, bv, bi, bp = lax.while_loop(
            scan_cond, scan_step,
            (0, ninf, maxi, jnp.zeros((16,), jnp.int32)))
        srtv, _ = plsc.sort_key_val(bv, bv)
        mval = _lane(srtv, 15)
        ci = jnp.where(bv == mval, bi, maxi)
        srti, _ = plsc.sort_key_val(ci, ci)
        mi = _lane(srti, 0)
        winner = ci == mi
        plsc.store_scatter(cvals, [bp], ninf, mask=winner)
        kvec = jnp.full((16,), 1, jnp.int32) * k
        lane0 = lanes == 0
        plsc.store_scatter(tkg, [kvec],
                           (mi & (_N // 2 - 1)) + row * (_N // 2), mask=lane0)
        plsc.store_scatter(thalf, [kvec], lax.shift_right_logical(mi, 12),
                           mask=lane0)
        return 0

    lax.fori_loop(0, _K, ext_body, 0)
    pltpu.async_copy(ve_hbm.at[tkg], wide, sem).wait()
    h0 = thalf[pl.ds(0, 16)]
    h1 = thalf[pl.ds(16, 16)]
    for j in range(_K * _H // 16):
        kj = j // (_H // 16)
        half = _lane(h0 if kj < 16 else h1, kj % 16)
        rowv = jnp.full((16,), kj, jnp.int32)
        colv = half * _H + (lanes + (j % (_H // 16)) * 16)
        rows[pl.ds(j * 16, 16)] = plsc.load_gather(wide, [rowv, colv])
    pltpu.sync_copy(rows, out_hbm.at[pl.ds(row * _K * _H, _K * _H)])


def _sc_topk_gather(compat, ve2):
    mesh = plsc.VectorSubcoreMesh(core_axis_name="c", subcore_axis_name="s")

    @functools.partial(
        pl.kernel, mesh=mesh,
        compiler_params=pltpu.CompilerParams(needs_layout_passes=False),
        out_type=jax.ShapeDtypeStruct((_R * _K * _H,), jnp.float32),
        scratch_types=[
            pltpu.VMEM((_N,), jnp.float32),
            pltpu.VMEM((_N + 16,), jnp.float32),
            pltpu.VMEM((_N + 16,), jnp.int32),
            pltpu.VMEM((_K,), jnp.int32),
            pltpu.VMEM((_K,), jnp.int32),
            pltpu.VMEM((_K, 2 * _H), jnp.float32),
            pltpu.VMEM((_K * _H,), jnp.float32),
            pltpu.SemaphoreType.DMA,
        ],
    )
    def k(compat_hbm, ve_hbm, out_hbm, crow, cvals, cidx, tkg, thalf, wide,
          rows, sem):
        wid = lax.axis_index("s") * 2 + lax.axis_index("c")
        for rr in range(2):
            _topk_row(compat_hbm, ve_hbm, out_hbm, crow, cvals, cidx, tkg,
                      thalf, wide, rows, sem, wid * 2 + rr)

    return k(compat, ve2)


def _final_body(vs_ref, g_ref, fvs_ref, fve_ref, fb_ref, out_ref):
    acc = jax.lax.dot_general(vs_ref[...], fvs_ref[...],
                              (((1,), (1,)), ((), ())), precision=_HP)
    acc = acc + jax.lax.dot_general(g_ref[...], fve_ref[...],
                                    (((1,), (1,)), ((), ())), precision=_HP)
    out_ref[...] = jnp.maximum(acc + fb_ref[...], 0.0)


def _final(vs2, g2, fwd_vs, fwd_ve, fwd_b2):
    return pl.pallas_call(
        _final_body,
        out_shape=jax.ShapeDtypeStruct((_R, _H), jnp.float32),
    )(vs2, g2, fwd_vs, fwd_ve, fwd_b2)


def kernel(vs, ve, ve_dead, Wq, Wk, Wv, mot_w, mot_b, fwd_w, fwd_b):
    vs3 = vs.reshape(_R, 1, _H)
    ve3 = ve.reshape(_R, _N, _H)
    dead3 = ve_dead.reshape(_R, 1, _N).astype(jnp.float32)
    compat3, vm3, wide3 = _pass1(vs3, ve3, dead3, Wq, Wk.astype(jnp.bfloat16),
                                 Wv, mot_w, mot_b.reshape(1, _H))
    gathered = _sc_topk_gather(compat3.reshape(_R * _N), wide3)

    g2 = gathered.reshape(_R, _K * _H)
    v_C = _final(vs3.reshape(_R, _H), g2, fwd_w[:, :_H], fwd_w[:, _H:],
                 fwd_b.reshape(1, _H))
    return (v_C.reshape(_B, _A, _H), vm3.reshape(_B, _A, _H))


# speedup vs baseline: 1.2590x; 1.0005x over previous
import functools
import math
import jax
import jax.numpy as jnp
from jax import lax
from jax.experimental import pallas as pl
from jax.experimental.pallas import tpu as pltpu
from jax.experimental.pallas import tpu_sc as plsc

_B, _A, _N, _H, _K = 16, 4, 8192, 64, 32
_R = _B * _A
_NORM = 1.0 / math.sqrt(_H)
_HP = jax.lax.Precision.HIGHEST


def _pass1_body(vs_ref, ve_ref, dead_ref, wq_ref, wk_ref, wv_ref, motw_ref,
                motb_ref, compat_ref, vm_ref, wide_ref):
    vsr = vs_ref[0]
    q = jax.lax.dot(vsr, wq_ref[...])
    qb = q.astype(jnp.bfloat16)
    veb = ve_ref[0]
    wide_ref[...] = jnp.concatenate([veb[:_N // 2], veb[_N // 2:]],
                                    axis=1).astype(jnp.float32)
    kb = jax.lax.dot(veb, wk_ref[...],
                     preferred_element_type=jnp.float32
                     ).astype(jnp.bfloat16)
    c = _NORM * jax.lax.dot_general(qb, kb, (((1,), (1,)), ((), ())),
                                    preferred_element_type=jnp.float32)
    c = jnp.where(dead_ref[0] != 0, -jnp.inf, c)
    compat_ref[0] = c
    m = jnp.max(c)
    e = jnp.exp(c - jnp.maximum(m, -1e30))
    s = jnp.sum(e)
    w = jax.lax.dot(e.astype(jnp.bfloat16), veb,
                    preferred_element_type=jnp.float32)
    inv = jnp.where(s > 0, 1.0 / s, 0.0)
    va = jax.lax.dot(w * inv, wv_ref[...], precision=_HP)
    vm_in = jnp.concatenate([vsr, va], axis=1)
    vm = jax.lax.dot_general(vm_in, motw_ref[...], (((1,), (1,)), ((), ())),
                             precision=_HP) + motb_ref[...]
    vm_ref[0] = jnp.maximum(vm, 0.0)


def _pass1(vs3, ve3, dead3, Wq, Wk, Wv, mot_w, mot_b2):
    return pl.pallas_call(
        _pass1_body,
        grid=(_R,),
        in_specs=[
            pl.BlockSpec((1, 1, _H), lambda r: (r, 0, 0)),
            pl.BlockSpec((1, _N, _H), lambda r: (r, 0, 0)),
            pl.BlockSpec((1, 1, _N), lambda r: (r, 0, 0)),
            pl.BlockSpec((_H, _H), lambda r: (0, 0)),
            pl.BlockSpec((_H, _H), lambda r: (0, 0)),
            pl.BlockSpec((_H, _H), lambda r: (0, 0)),
            pl.BlockSpec((_H, 2 * _H), lambda r: (0, 0)),
            pl.BlockSpec((1, _H), lambda r: (0, 0)),
        ],
        out_specs=[
            pl.BlockSpec((1, 1, _N), lambda r: (r, 0, 0)),
            pl.BlockSpec((1, 1, _H), lambda r: (r, 0, 0)),
            pl.BlockSpec((_N // 2, 2 * _H), lambda r: (r, 0)),
        ],
        out_shape=[
            jax.ShapeDtypeStruct((_R, 1, _N), jnp.float32),
            jax.ShapeDtypeStruct((_R, 1, _H), jnp.float32),
            jax.ShapeDtypeStruct((_R * _N // 2, 2 * _H), jnp.float32),
        ],
    )(vs3, ve3, dead3, Wq, Wk, Wv, mot_w, mot_b2)


_NV = _N // 16
_MAXI = 2**31 - 1
_NEG = float("-inf")


_GD = lax.GatherDimensionNumbers(offset_dims=(), collapsed_slice_dims=(0,),
                                 start_index_map=(0,))


def _lane(x, j):
    return lax.gather(x, jnp.full((16, 1), j, jnp.int32), _GD,
                      slice_sizes=(1,),
                      mode=lax.GatherScatterMode.PROMISE_IN_BOUNDS)


def _topk_row(compat_hbm, ve_hbm, gw_hbm, half_hbm, crow, cvals,
              cidx, tkg, thalf, wide, sem, row):
    pltpu.sync_copy(compat_hbm.at[pl.ds(row * _N, _N)], crow)
    lanes = lax.iota(jnp.int32, 16)
    ninf = jnp.full((16,), _NEG, jnp.float32)
    maxi = jnp.full((16,), _MAXI, jnp.int32)

    def t2_body(i, carry):
        t1, t2 = carry
        v = crow[pl.ds(i * 16, 16)]
        m = v > t1
        hi = jnp.where(m, v, t1)
        lo = jnp.where(m, t1, v)
        return hi, jnp.maximum(t2, lo)

    _, t2 = lax.fori_loop(0, _NV, t2_body, (ninf, ninf))
    srt2, _ = plsc.sort_key_val(t2, t2)
    t = _lane(srt2, 0)

    def comp_body(i, off):
        v = crow[pl.ds(i * 16, 16)]
        mask = v >= t
        pc = plsc.cumsum(mask.astype(jnp.int32))
        pos = jnp.where(mask, off + pc - 1, 0)
        plsc.store_scatter(cvals, [pos], v, mask=mask)
        plsc.store_scatter(cidx, [pos], lanes + i * 16, mask=mask)
        return off + _lane(pc, 15)

    off = lax.fori_loop(0, _NV, comp_body, jnp.zeros((16,), jnp.int32))
    plsc.store_scatter(cvals, [off + lanes], ninf)

    def ext_body(k, _):
        def scan_cond(c):
            return jnp.any(c[0] * 16 < off)

        def scan_step(c):
            j, bv, bi, bp = c
            v = cvals[pl.ds(j * 16, 16)]
            iv = cidx[pl.ds(j * 16, 16)]
            pos = lanes + j * 16
            m = (v > bv) | ((v == bv) & (iv < bi))
            return (j + 1, jnp.where(m, v, bv), jnp.where(m, iv, bi),
                    jnp.where(m, pos, bp))

        _, bv, bi, bp = lax.while_loop(
            scan_cond, scan_step,
            (0, ninf, maxi, jnp.zeros((16,), jnp.int32)))
        srtv, _ = plsc.sort_key_val(bv, bv)
        mval = _lane(srtv, 15)
        ci = jnp.where(bv == mval, bi, maxi)
        srti, _ = plsc.sort_key_val(ci, ci)
        mi = _lane(srti, 0)
        winner = ci == mi
        plsc.store_scatter(cvals, [bp], ninf, mask=winner)
        kvec = jnp.full((16,), 1, jnp.int32) * k
        lane0 = lanes == 0
        plsc.store_scatter(tkg, [kvec],
                           (mi & (_N // 2 - 1)) + row * (_N // 2), mask=lane0)
        plsc.store_scatter(thalf, [kvec], lax.shift_right_logical(mi, 12),
                           mask=lane0)
        return 0

    lax.fori_loop(0, _K, ext_body, 0)
    pltpu.async_copy(ve_hbm.at[tkg], wide, sem).wait()
    pltpu.sync_copy(wide, gw_hbm.at[row])
    pltpu.sync_copy(thalf, half_hbm.at[row])


def _sc_topk_gather(compat, ve2):
    mesh = plsc.VectorSubcoreMesh(core_axis_name="c", subcore_axis_name="s")

    @functools.partial(
        pl.kernel, mesh=mesh,
        compiler_params=pltpu.CompilerParams(needs_layout_passes=False),
        out_type=[
            jax.ShapeDtypeStruct((_R, _K, 2 * _H), jnp.float32),
            jax.ShapeDtypeStruct((_R, _K), jnp.int32),
        ],
        scratch_types=[
            pltpu.VMEM((_N,), jnp.float32),
            pltpu.VMEM((_N + 16,), jnp.float32),
            pltpu.VMEM((_N + 16,), jnp.int32),
            pltpu.VMEM((_K,), jnp.int32),
            pltpu.VMEM((_K,), jnp.int32),
            pltpu.VMEM((_K, 2 * _H), jnp.float32),
            pltpu.SemaphoreType.DMA,
        ],
    )
    def k(compat_hbm, ve_hbm, gw_hbm, half_hbm, crow, cvals, cidx,
          tkg, thalf, wide, sem):
        wid = lax.axis_index("s") * 2 + lax.axis_index("c")
        for rr in range(2):
            _topk_row(compat_hbm, ve_hbm, gw_hbm, half_hbm, crow,
                      cvals, cidx, tkg, thalf, wide, sem, wid * 2 + rr)

    return k(compat, ve2)


def _final_body(vs_ref, gw_ref, hv_ref, fvs_ref, fve_ref, fb_ref, out_ref):
    acc = jax.lax.dot_general(vs_ref[...], fvs_ref[...],
                              (((1,), (1,)), ((), ())), precision=_HP)
    for k in range(_K):
        sel = hv_ref[:, k:k + 1] != 0
        gk = jnp.where(sel, gw_ref[:, k, _H:], gw_ref[:, k, :_H])
        acc = acc + jax.lax.dot(gk, fve_ref[k])
    out_ref[...] = jnp.maximum(acc + fb_ref[...], 0.0)


def _final(vs2, gw, hv, fwd_vs, fwd_ve3, fwd_b2):
    return pl.pallas_call(
        _final_body,
        out_shape=jax.ShapeDtypeStruct((_R, _H), jnp.float32),
    )(vs2, gw, hv, fwd_vs, fwd_ve3, fwd_b2)


def kernel(vs, ve, ve_dead, Wq, Wk, Wv, mot_w, mot_b, fwd_w, fwd_b):
    vs3 = vs.reshape(_R, 1, _H)
    ve3b = ve.astype(jnp.bfloat16).reshape(_R, _N, _H)
    dead3 = ve_dead.reshape(_R, 1, _N).astype(jnp.float32)
    compat3, vm3, wide3 = _pass1(vs3, ve3b, dead3, Wq,
                                 Wk.astype(jnp.bfloat16), Wv, mot_w,
                                 mot_b.reshape(1, _H))
    gw, hv = _sc_topk_gather(compat3.reshape(_R * _N), wide3)

    fwd_ve3 = fwd_w[:, _H:].T.reshape(_K, _H, _H)
    v_C = _final(vs3.reshape(_R, _H), gw, hv, fwd_w[:, :_H], fwd_ve3,
                 fwd_b.reshape(1, _H))
    return (v_C.reshape(_B, _A, _H), vm3.reshape(_B, _A, _H))
